# baseline (speedup 1.0000x reference)
import functools

import jax
import jax.numpy as jnp
from jax import lax
from jax.experimental import pallas as pl
from jax.experimental.pallas import tpu as pltpu
from jax.experimental.pallas import tpu_sc as plsc

N_VALUES = 1000
LANES = 16
CHUNK = 128


@functools.cache
def _make_planes(V, D):
    pv = 128 * (-(-V // 128))
    va = V - V % 128

    def body(in_ref, out_ref, sem):
        copies = [
            pltpu.make_async_copy(
                in_ref.at[j, pl.ds(0, va)], out_ref.at[pl.ds(j * pv, va)],
                sem)
            for j in range(D)
        ]
        for cp in copies:
            cp.start()
        for cp in copies:
            cp.wait()

    return pl.pallas_call(
        body,
        in_specs=[pl.BlockSpec(memory_space=pl.ANY)],
        out_specs=pl.BlockSpec(memory_space=pl.ANY),
        out_shape=jax.ShapeDtypeStruct((D * pv,), jnp.int32),
        scratch_shapes=[pltpu.SemaphoreType.DMA],
    )


@functools.cache
def _make_gather(B, V, D):
    info = plsc.get_sparse_core_info()
    nw = info.num_cores * info.num_subcores
    b_per_w = B // nw
    n_chunks = b_per_w // CHUNK
    mesh = plsc.VectorSubcoreMesh(core_axis_name="c", subcore_axis_name="s")

    pv = 128 * (-(-V // 128))
    va = V - V % 128

    @functools.partial(
        pl.kernel,
        mesh=mesh,
        out_type=jax.ShapeDtypeStruct((B * 8,), jnp.int32),
        compiler_params=pltpu.CompilerParams(
            use_tc_tiling_on_sc=False, needs_layout_passes=False),
        scratch_types=[
            pltpu.VMEM((b_per_w,), jnp.int32),
            pltpu.VMEM((b_per_w,), jnp.int32),
            pltpu.VMEM((D * n_chunks, CHUNK), jnp.int32),
            pltpu.VMEM((D, b_per_w), jnp.int32),
            pltpu.VMEM((b_per_w * 8,), jnp.int32),
            pltpu.VMEM((512,), jnp.int32),
            pltpu.SemaphoreType.DMA,
        ],
    )
    def gather_kernel(x_hbm, sh_hbm, planes_hbm, tail_hbm, out_hbm,
                      x_v, sh_v, idx_v, stage_v, rows_v, tail_v, sem):
        wid = lax.axis_index("s") * info.num_cores + lax.axis_index("c")
        base = wid * b_per_w
        pltpu.sync_copy(x_hbm.at[pl.ds(base, b_per_w)], x_v)
        pltpu.sync_copy(sh_hbm.at[pl.ds(base, b_per_w)], sh_v)
        pltpu.sync_copy(tail_hbm, tail_v)
        for c in range(n_chunks):
            for v in range(CHUNK // LANES):
                sl = pl.ds(c * CHUNK + v * LANES, LANES)
                idx = x_v[sl] * N_VALUES + sh_v[sl]
                for j in range(D):
                    idx_v[j * n_chunks + c, pl.ds(v * LANES, LANES)] = (
                        idx + j * pv)
        copies = []
        for j in range(D):
            for c in range(n_chunks):
                cp = pltpu.make_async_copy(
                    planes_hbm.at[idx_v.at[j * n_chunks + c]],
                    stage_v.at[j, pl.ds(c * CHUNK, CHUNK)], sem)
                cp.start()
                copies.append(cp)
        for cp in copies:
            cp.wait()
        lane8 = lax.iota(jnp.int32, LANES) * 8
        for j in range(D):
            for v in range(b_per_w // LANES):
                i_vec = idx_v[v // 8, pl.ds((v % 8) * LANES, LANES)]
                in_tail = i_vec >= va
                tpos = jnp.maximum(i_vec - va, 0) + j * 64
                tvals = plsc.load_gather(tail_v, [tpos])
                vals = jnp.where(
                    in_tail, tvals, stage_v[j, pl.ds(v * LANES, LANES)]) + 1
                plsc.store_scatter(rows_v, [lane8 + (v * LANES * 8 + j)], vals)
        pltpu.sync_copy(rows_v, out_hbm.at[pl.ds(base * 8, b_per_w * 8)])

    return gather_kernel


def kernel(x, mapping):
    batch = x.shape[0]
    V, D = mapping.shape
    key = jax.random.key(42)
    shifts = jnp.stack(
        [jax.random.randint(jax.random.fold_in(key, i), (batch,), 0, N_VALUES,
                            dtype=x.dtype) for i in range(2)],
        axis=1)
    planes = _make_planes(V, D)(mapping.T)
    va = V - V % 128
    tail = jnp.pad(mapping[va:, :].T.reshape(-1), (0, 512 - (V - va) * D))
    out8 = _make_gather(2 * batch, V, D)(
        x.reshape(-1), shifts.reshape(-1), planes, tail).reshape(2 * batch, 8)
    result = out8[:, :D].reshape(batch, 2 * D)
    zeros = jnp.zeros(result.shape, jnp.float32)
    return (result, zeros, zeros)


# speedup vs baseline: 6.6466x; 6.6466x over previous
import functools

import jax
import jax.numpy as jnp
from jax import lax
from jax.experimental import pallas as pl
from jax.experimental.pallas import tpu as pltpu
from jax.experimental.pallas import tpu_sc as plsc

N_VALUES = 1000
LANES = 16
CHUNK = 128


@functools.cache
def _make_planes(V, D):
    info = plsc.get_sparse_core_info()
    nw = info.num_cores * info.num_subcores
    va = V - V % 128
    nt = va // 128
    stride = nt // nw
    per_w = nt - (nw - 1) * stride
    n_sub = 4
    sub = -(-per_w // n_sub)
    wc = sub * 128
    mesh = plsc.VectorSubcoreMesh(core_axis_name="c", subcore_axis_name="s")

    @functools.partial(
        pl.kernel,
        mesh=mesh,
        out_type=jax.ShapeDtypeStruct((D * va,), jnp.int32),
        compiler_params=pltpu.CompilerParams(
            use_tc_tiling_on_sc=True, needs_layout_passes=False),
        scratch_types=[
            pltpu.VMEM((D, wc), jnp.int32),
            pltpu.SemaphoreType.DMA,
        ],
    )
    def detile_kernel(table_hbm, planes_hbm, blk_v, sem):
        wid = lax.axis_index("s") * info.num_cores + lax.axis_index("c")
        start = wid * stride
        for k in range(n_sub):
            t0 = jnp.minimum(start + k * sub, nt - sub)
            col = pl.multiple_of(t0 * 128, 128)
            pltpu.sync_copy(table_hbm.at[:, pl.ds(col, wc)], blk_v)
            for j in range(D):
                pltpu.sync_copy(
                    blk_v.at[j],
                    planes_hbm.at[pl.ds(j * va + col, wc)])

    return detile_kernel


@functools.cache
def _make_gather(B, V, D):
    info = plsc.get_sparse_core_info()
    nw = info.num_cores * info.num_subcores
    b_per_w = B // nw
    n_chunks = b_per_w // CHUNK
    va = V - V % 128
    mesh = plsc.VectorSubcoreMesh(core_axis_name="c", subcore_axis_name="s")

    @functools.partial(
        pl.kernel,
        mesh=mesh,
        out_type=jax.ShapeDtypeStruct((B * 8,), jnp.int32),
        compiler_params=pltpu.CompilerParams(
            use_tc_tiling_on_sc=False, needs_layout_passes=False),
        scratch_types=[
            pltpu.VMEM((b_per_w,), jnp.int32),
            pltpu.VMEM((b_per_w,), jnp.int32),
            pltpu.VMEM((D * n_chunks, CHUNK), jnp.int32),
            pltpu.VMEM((D, b_per_w), jnp.int32),
            pltpu.VMEM((b_per_w * 8,), jnp.int32),
            pltpu.VMEM((512,), jnp.int32),
            pltpu.SemaphoreType.DMA,
        ],
    )
    def gather_kernel(x_hbm, sh_hbm, planes_hbm, tail_hbm, out_hbm,
                      x_v, sh_v, idx_v, stage_v, rows_v, tail_v, sem):
        wid = lax.axis_index("s") * info.num_cores + lax.axis_index("c")
        base = wid * b_per_w
        pltpu.sync_copy(x_hbm.at[pl.ds(base, b_per_w)], x_v)
        pltpu.sync_copy(sh_hbm.at[pl.ds(base, b_per_w)], sh_v)
        pltpu.sync_copy(tail_hbm, tail_v)
        for c in range(n_chunks):
            for v in range(CHUNK // LANES):
                sl = pl.ds(c * CHUNK + v * LANES, LANES)
                idx = x_v[sl] * N_VALUES + sh_v[sl]
                dst = pl.ds(v * LANES, LANES)
                idx_v[c, dst] = idx
                clamped = jnp.minimum(idx, va - 1)
                for j in range(1, D):
                    idx_v[j * n_chunks + c, dst] = clamped + j * va
        copies = []
        for j in range(D):
            for c in range(n_chunks):
                cp = pltpu.make_async_copy(
                    planes_hbm.at[idx_v.at[j * n_chunks + c]],
                    stage_v.at[j, pl.ds(c * CHUNK, CHUNK)], sem)
                cp.start()
                copies.append(cp)
        for cp in copies:
            cp.wait()
        lane8 = lax.iota(jnp.int32, LANES) * 8
        for j in range(D):
            for v in range(b_per_w // LANES):
                i_vec = idx_v[v // 8, pl.ds((v % 8) * LANES, LANES)]
                in_tail = i_vec >= va
                tpos = jnp.maximum(i_vec - va, 0) + j * 64
                tvals = plsc.load_gather(tail_v, [tpos])
                vals = jnp.where(
                    in_tail, tvals, stage_v[j, pl.ds(v * LANES, LANES)]) + 1
                plsc.store_scatter(rows_v, [lane8 + (v * LANES * 8 + j)], vals)
        pltpu.sync_copy(rows_v, out_hbm.at[pl.ds(base * 8, b_per_w * 8)])

    return gather_kernel


def kernel(x, mapping):
    batch = x.shape[0]
    V, D = mapping.shape
    key = jax.random.key(42)
    shifts = jnp.stack(
        [jax.random.randint(jax.random.fold_in(key, i), (batch,), 0, N_VALUES,
                            dtype=x.dtype) for i in range(2)],
        axis=1)
    planes = _make_planes(V, D)(mapping.T)
    va = V - V % 128
    tail = jnp.pad(mapping[va:, :].T.reshape(-1), (0, 512 - (V - va) * D))
    out8 = _make_gather(2 * batch, V, D)(
        x.reshape(-1), shifts.reshape(-1), planes, tail).reshape(2 * batch, 8)
    result = out8[:, :D].reshape(batch, 2 * D)
    zeros = jnp.zeros(result.shape, jnp.float32)
    return (result, zeros, zeros)


# speedup vs baseline: 7.4160x; 1.1157x over previous
import functools

import jax
import jax.numpy as jnp
from jax import lax
from jax.experimental import pallas as pl
from jax.experimental.pallas import tpu as pltpu
from jax.experimental.pallas import tpu_sc as plsc

N_VALUES = 1000
LANES = 16
CHUNK = 128


@functools.cache
def _make_planes(V, D):
    info = plsc.get_sparse_core_info()
    nw = info.num_cores * info.num_subcores
    va = V - V % 128
    nt = va // 128
    stride = nt // nw
    per_w = nt - (nw - 1) * stride
    n_sub = 2
    sub = -(-per_w // n_sub)
    wc = sub * 128
    mesh = plsc.VectorSubcoreMesh(core_axis_name="c", subcore_axis_name="s")

    @functools.partial(
        pl.kernel,
        mesh=mesh,
        out_type=jax.ShapeDtypeStruct((D * va,), jnp.int32),
        compiler_params=pltpu.CompilerParams(
            use_tc_tiling_on_sc=True, needs_layout_passes=False),
        scratch_types=[
            pltpu.VMEM((D, wc), jnp.int32),
            pltpu.SemaphoreType.DMA,
        ],
    )
    def detile_kernel(table_hbm, planes_hbm, blk_v, sem):
        wid = lax.axis_index("s") * info.num_cores + lax.axis_index("c")
        start = wid * stride
        for k in range(n_sub):
            t0 = jnp.minimum(start + k * sub, nt - sub)
            col = pl.multiple_of(t0 * 128, 128)
            pltpu.sync_copy(table_hbm.at[:, pl.ds(col, wc)], blk_v)
            for j in range(D):
                pltpu.sync_copy(
                    blk_v.at[j],
                    planes_hbm.at[pl.ds(j * va + col, wc)])

    return detile_kernel


@functools.cache
def _make_gather(B, V, D):
    info = plsc.get_sparse_core_info()
    nw = info.num_cores * info.num_subcores
    b_per_w = B // nw
    n_chunks = b_per_w // CHUNK
    va = V - V % 128
    mesh = plsc.VectorSubcoreMesh(core_axis_name="c", subcore_axis_name="s")

    @functools.partial(
        pl.kernel,
        mesh=mesh,
        out_type=jax.ShapeDtypeStruct((B * D,), jnp.int32),
        compiler_params=pltpu.CompilerParams(
            use_tc_tiling_on_sc=False, needs_layout_passes=False),
        scratch_types=[
            pltpu.VMEM((b_per_w,), jnp.int32),
            pltpu.VMEM((b_per_w,), jnp.int32),
            pltpu.VMEM((D * n_chunks, CHUNK), jnp.int32),
            pltpu.VMEM((D, b_per_w), jnp.int32),
            pltpu.VMEM((b_per_w * D,), jnp.int32),
            pltpu.VMEM((512,), jnp.int32),
            pltpu.SemaphoreType.DMA,
        ],
    )
    def gather_kernel(x_hbm, sh_hbm, planes_hbm, tail_hbm, out_hbm,
                      x_v, sh_v, idx_v, stage_v, rows_v, tail_v, sem):
        wid = lax.axis_index("s") * info.num_cores + lax.axis_index("c")
        base = wid * b_per_w
        pltpu.sync_copy(x_hbm.at[pl.ds(base, b_per_w)], x_v)
        pltpu.sync_copy(sh_hbm.at[pl.ds(base, b_per_w)], sh_v)
        pltpu.sync_copy(tail_hbm, tail_v)
        for c in range(n_chunks):
            for v in range(CHUNK // LANES):
                sl = pl.ds(c * CHUNK + v * LANES, LANES)
                idx = x_v[sl] * N_VALUES + sh_v[sl]
                dst = pl.ds(v * LANES, LANES)
                idx_v[c, dst] = idx
                clamped = jnp.minimum(idx, va - 1)
                for j in range(1, D):
                    idx_v[j * n_chunks + c, dst] = clamped + j * va
        copies = []
        for j in range(D):
            for c in range(n_chunks):
                cp = pltpu.make_async_copy(
                    planes_hbm.at[idx_v.at[j * n_chunks + c]],
                    stage_v.at[j, pl.ds(c * CHUNK, CHUNK)], sem)
                cp.start()
                copies.append(cp)
        for cp in copies:
            cp.wait()
        laneD = lax.iota(jnp.int32, LANES) * D
        for j in range(D):
            for v in range(b_per_w // LANES):
                i_vec = idx_v[v // 8, pl.ds((v % 8) * LANES, LANES)]
                in_tail = i_vec >= va
                tpos = jnp.maximum(i_vec - va, 0) + j * 64
                tvals = plsc.load_gather(tail_v, [tpos])
                vals = jnp.where(
                    in_tail, tvals, stage_v[j, pl.ds(v * LANES, LANES)]) + 1
                plsc.store_scatter(rows_v, [laneD + (v * LANES * D + j)], vals)
        pltpu.sync_copy(rows_v, out_hbm.at[pl.ds(base * D, b_per_w * D)])

    return gather_kernel


def kernel(x, mapping):
    batch = x.shape[0]
    V, D = mapping.shape
    key = jax.random.key(42)
    shifts = jnp.stack(
        [jax.random.randint(jax.random.fold_in(key, i), (batch,), 0, N_VALUES,
                            dtype=x.dtype) for i in range(2)],
        axis=1)
    planes = _make_planes(V, D)(mapping.T)
    va = V - V % 128
    tail = jnp.pad(mapping[va:, :].T.reshape(-1), (0, 512 - (V - va) * D))
    flat = _make_gather(2 * batch, V, D)(
        x.reshape(-1), shifts.reshape(-1), planes, tail)
    result = flat.reshape(batch, 2 * D)
    zeros = jnp.zeros(result.shape, jnp.float32)
    return (result, zeros, zeros)
